# Initial kernel scaffold; baseline (speedup 1.0000x reference)
import jax
import jax.numpy as jnp
from jax.experimental import pallas as pl


def kernel(x, edge_index, W_pre, W_nn, bias):
    raise NotImplementedError("write your pallas kernel here")


# speedup vs baseline: 4.7254x; 4.7254x over previous
import functools

import jax
import jax.numpy as jnp
from jax import lax
from jax.experimental import pallas as pl
from jax.experimental.pallas import tpu as pltpu
from jax.experimental.pallas import tpu_sc as plsc

N = 10000
K = 32
C = 128
NW = 32
PW = 320
NPAD = NW * PW
CH = 4
G = CH * K
CHUNKS = PW // CH
BLK = 1024

_HI = jax.lax.Precision.HIGHEST


def _mm_relu_body(x_ref, w_ref, o_ref):
    o_ref[...] = jnp.maximum(
        jnp.dot(x_ref[...], w_ref[...], preferred_element_type=jnp.float32,
                precision=_HI), 0.0)


def _pre_mlp(xt_pad, w_pre_t):
    return pl.pallas_call(
        _mm_relu_body,
        grid=(NPAD // BLK,),
        in_specs=[
            pl.BlockSpec((BLK, C), lambda i: (i, 0)),
            pl.BlockSpec((C, C), lambda i: (0, 0)),
        ],
        out_specs=pl.BlockSpec((BLK, C), lambda i: (i, 0)),
        out_shape=jax.ShapeDtypeStruct((NPAD, C), jnp.float32),
    )(xt_pad, w_pre_t)


def _post_body(x_ref, a_ref, w1_ref, w2_ref, b_ref, o_ref):
    h = (jnp.dot(x_ref[...], w1_ref[...], preferred_element_type=jnp.float32,
                 precision=_HI)
         + jnp.dot(a_ref[...], w2_ref[...], preferred_element_type=jnp.float32,
                   precision=_HI))
    h = jnp.maximum(h, 0.0) + b_ref[...]
    nrm = jnp.sqrt(jnp.sum(h * h, axis=1, keepdims=True))
    o_ref[...] = h / jnp.maximum(nrm, 1e-12)


def _post_mlp(xt_pad, aggr, w1_t, w2_t, b_row):
    return pl.pallas_call(
        _post_body,
        grid=(NPAD // BLK,),
        in_specs=[
            pl.BlockSpec((BLK, C), lambda i: (i, 0)),
            pl.BlockSpec((BLK, C), lambda i: (i, 0)),
            pl.BlockSpec((C, C), lambda i: (0, 0)),
            pl.BlockSpec((C, C), lambda i: (0, 0)),
            pl.BlockSpec((1, C), lambda i: (0, 0)),
        ],
        out_specs=pl.BlockSpec((BLK, C), lambda i: (i, 0)),
        out_shape=jax.ShapeDtypeStruct((NPAD, C), jnp.float32),
    )(xt_pad, aggr, w1_t, w2_t, b_row)


def _gather_max_sc(idx_grp, y):
    mesh = plsc.VectorSubcoreMesh(core_axis_name="c", subcore_axis_name="s")

    @functools.partial(
        pl.kernel,
        mesh=mesh,
        out_type=jax.ShapeDtypeStruct((NPAD, C), jnp.float32),
        scratch_types=[
            pltpu.VMEM((CHUNKS, G), jnp.int32),
            pltpu.VMEM((G, C), jnp.float32),
            pltpu.VMEM((PW, C), jnp.float32),
            pltpu.SemaphoreType.DMA,
        ],
    )
    def sc_kernel(idx_hbm, y_hbm, out_hbm, idx_v, rows_v, out_v, sem):
        wid = lax.axis_index("s") * 2 + lax.axis_index("c")
        pltpu.sync_copy(idx_hbm.at[wid], idx_v)

        def chunk_body(c, carry):
            pltpu.async_copy(y_hbm.at[idx_v.at[c]], rows_v, sem).wait()
            for n in range(CH):
                for g in range(C // 16):
                    sl = pl.ds(g * 16, 16)
                    vals = [rows_v[n * K + k, sl] for k in range(K)]
                    while len(vals) > 1:
                        vals = [jnp.maximum(vals[2 * i], vals[2 * i + 1])
                                for i in range(len(vals) // 2)]
                    out_v[c * CH + n, sl] = vals[0]
            return carry

        lax.fori_loop(0, CHUNKS, chunk_body, 0)
        pltpu.sync_copy(out_v, out_hbm.at[pl.ds(wid * PW, PW)])

    return sc_kernel(idx_grp, y)


def kernel(x, edge_index, W_pre, W_nn, bias):
    xt = x.reshape(C, N).T
    xt_pad = jnp.pad(xt, ((0, NPAD - N), (0, 0)))
    idx = edge_index[0].reshape(N, K)
    idx_grp = jnp.pad(idx, ((0, NPAD - N), (0, 0))).reshape(NW, CHUNKS, G)

    y = _pre_mlp(xt_pad, W_pre.T)
    aggr = _gather_max_sc(idx_grp, y)
    out = _post_mlp(xt_pad, aggr, W_nn[:, :C].T, W_nn[:, C:].T,
                    bias.reshape(1, C))
    return out[:N].T.reshape(1, C, N, 1)
